# Initial kernel scaffold; baseline (speedup 1.0000x reference)
import functools

import jax
import jax.numpy as jnp
from jax import lax
from jax.experimental import pallas as pl
from jax.experimental.pallas import tpu as pltpu
from jax.experimental.pallas import tpu_sc as plsc

N_VERT = 10000
N_HE = 320000
D = 128

NC = 2
NS = 16
NW = NC * NS
HE_PER_W = N_HE // NW
G = 125
GROUPS = HE_PER_W // G
CHUNK_G = 16
N_CHUNKS = GROUPS // CHUNK_G
V_PER_TILE = N_VERT // NS
V_CHUNKS = V_PER_TILE // G


def _sc_partial_sums(x, idx3, zrows):
    mesh = plsc.VectorSubcoreMesh(core_axis_name="c", subcore_axis_name="s")

    @functools.partial(
        pl.kernel,
        out_type=jax.ShapeDtypeStruct((NC, N_VERT, D), jnp.float32),
        mesh=mesh,
        scratch_types=[
            pltpu.VMEM((CHUNK_G, G), jnp.int32),
            pltpu.VMEM((G, D), jnp.float32),
            pltpu.VMEM((G, D), jnp.float32),
            pltpu.VMEM_SHARED((N_VERT, D), jnp.float32),
        ],
    )
    def k(x_hbm, idx_hbm, z_hbm, out_hbm, idx_v, buf0, buf1, acc):
        c = lax.axis_index("c")
        s = lax.axis_index("s")
        wid = c * NS + s
        he_base = wid * HE_PER_W

        pltpu.sync_copy(z_hbm, buf0)
        for kk in range(V_CHUNKS):
            pltpu.sync_copy(buf0, acc.at[pl.ds(s * V_PER_TILE + kk * G, G)])
        plsc.subcore_barrier()

        def chunk_body(ci, _):
            pltpu.sync_copy(idx_hbm.at[wid, pl.ds(ci * CHUNK_G, CHUNK_G)], idx_v)

            def group_body(g, _):
                row0 = he_base + (ci * CHUNK_G + g) * G
                pltpu.sync_copy(x_hbm.at[pl.ds(row0, G)], buf0)
                pltpu.sync_copy(buf0, acc.at[idx_v.at[g]], add=True)
                return 0

            lax.fori_loop(0, CHUNK_G, group_body, 0)
            return 0

        lax.fori_loop(0, N_CHUNKS, chunk_body, 0)
        plsc.subcore_barrier()

        def out_body(kk, _):
            r0 = s * V_PER_TILE + kk * G
            pltpu.sync_copy(acc.at[pl.ds(r0, G)], buf1)
            pltpu.sync_copy(buf1, out_hbm.at[c, pl.ds(r0, G)])
            return 0

        lax.fori_loop(0, V_CHUNKS, out_body, 0)

    return k(x, idx3, zrows)


def _combine_body(p_ref, v_ref, o_ref):
    o_ref[...] = (p_ref[0] + p_ref[1]) / v_ref[...]


def _combine(partials, valence):
    rb = 1000
    grid = N_VERT // rb
    return pl.pallas_call(
        _combine_body,
        grid=(grid,),
        in_specs=[
            pl.BlockSpec((NC, rb, D), lambda i: (0, i, 0)),
            pl.BlockSpec((rb, 1), lambda i: (i, 0)),
        ],
        out_specs=pl.BlockSpec((rb, D), lambda i: (i, 0)),
        out_shape=jax.ShapeDtypeStruct((N_VERT, D), jnp.float32),
    )(partials, valence.reshape(N_VERT, 1))


def kernel(x, half_edge_src, vertex_valence):
    idx3 = half_edge_src.astype(jnp.int32).reshape(NW, GROUPS, G)
    zrows = jnp.zeros((G, D), jnp.float32)
    partials = _sc_partial_sums(x, idx3, zrows)
    return _combine(partials, vertex_valence)


# speedup vs baseline: 4.4993x; 4.4993x over previous
import functools

import jax
import jax.numpy as jnp
from jax import lax
from jax.experimental import pallas as pl
from jax.experimental.pallas import tpu as pltpu
from jax.experimental.pallas import tpu_sc as plsc

N_VERT = 10000
N_HE = 320000
D = 128

NC = 2
NS = 16
NW = NC * NS
HE_PER_W = N_HE // NW
G = 80
GROUPS = HE_PER_W // G
V_CHUNKS = N_VERT // G
RR = (V_CHUNKS + NS - 1) // NS


def _sc_partial_sums(x, idx3, zrows):
    mesh = plsc.VectorSubcoreMesh(core_axis_name="c", subcore_axis_name="s")

    @functools.partial(
        pl.kernel,
        out_type=jax.ShapeDtypeStruct((NC, N_VERT, D), jnp.float32),
        mesh=mesh,
        scratch_types=[
            pltpu.VMEM((GROUPS, G), jnp.int32),
            pltpu.VMEM((G, D), jnp.float32),
            pltpu.VMEM((G, D), jnp.float32),
            pltpu.VMEM_SHARED((N_VERT, D), jnp.float32),
        ],
    )
    def k(x_hbm, idx_hbm, z_hbm, out_hbm, idx_v, buf0, buf1, acc):
        c = lax.axis_index("c")
        s = lax.axis_index("s")
        wid = c * NS + s
        he_base = wid * HE_PER_W

        pltpu.sync_copy(idx_hbm.at[wid], idx_v)

        pltpu.sync_copy(z_hbm, buf0)
        for j in range(RR):
            cid = s + j * NS

            @pl.when(cid < V_CHUNKS)
            def _():
                pltpu.sync_copy(buf0, acc.at[pl.ds(cid * G, G)])
        plsc.subcore_barrier()

        def group_body(g, _):
            pltpu.sync_copy(x_hbm.at[pl.ds(he_base + g * G, G)], buf0)
            pltpu.sync_copy(buf0, acc.at[idx_v.at[g]], add=True)
            return 0

        lax.fori_loop(0, GROUPS, group_body, 0)
        plsc.subcore_barrier()

        def out_chunk(cid):
            pltpu.sync_copy(acc.at[pl.ds(cid * G, G)], buf1)
            pltpu.sync_copy(buf1, out_hbm.at[c, pl.ds(cid * G, G)])

        for j in range(RR):
            cid = s + j * NS
            @pl.when(cid < V_CHUNKS)
            def _():
                out_chunk(cid)

    return k(x, idx3, zrows)


def _combine_body(p_ref, v_ref, o_ref):
    o_ref[...] = (p_ref[0] + p_ref[1]) / v_ref[...]


def _combine(partials, valence):
    rb = 1000
    grid = N_VERT // rb
    return pl.pallas_call(
        _combine_body,
        grid=(grid,),
        in_specs=[
            pl.BlockSpec((NC, rb, D), lambda i: (0, i, 0)),
            pl.BlockSpec((rb, 1), lambda i: (i, 0)),
        ],
        out_specs=pl.BlockSpec((rb, D), lambda i: (i, 0)),
        out_shape=jax.ShapeDtypeStruct((N_VERT, D), jnp.float32),
    )(partials, valence.reshape(N_VERT, 1))


def kernel(x, half_edge_src, vertex_valence):
    idx3 = half_edge_src.astype(jnp.int32).reshape(NW, GROUPS, G)
    zrows = jnp.zeros((G, D), jnp.float32)
    partials = _sc_partial_sums(x, idx3, zrows)
    return _combine(partials, vertex_valence)


# speedup vs baseline: 7.7354x; 1.7192x over previous
import functools

import jax
import jax.numpy as jnp
from jax import lax
from jax.experimental import pallas as pl
from jax.experimental.pallas import tpu as pltpu
from jax.experimental.pallas import tpu_sc as plsc

N_VERT = 10000
N_HE = 320000
D = 128

NC = 2
NS = 16
NW = NC * NS
HE_PER_W = N_HE // NW
G = 80
GROUPS = HE_PER_W // G
V_CHUNKS = N_VERT // G
RR = (V_CHUNKS + NS - 1) // NS
NBUF = 3


def _sc_partial_sums(x, idx3, zrows):
    mesh = plsc.VectorSubcoreMesh(core_axis_name="c", subcore_axis_name="s")

    @functools.partial(
        pl.kernel,
        out_type=jax.ShapeDtypeStruct((NC, N_VERT, D), jnp.float32),
        mesh=mesh,
        scratch_types=[
            pltpu.VMEM((GROUPS, G), jnp.int32),
            [pltpu.VMEM((G, D), jnp.float32) for _ in range(NBUF)],
            pltpu.VMEM_SHARED((N_VERT, D), jnp.float32),
            [pltpu.SemaphoreType.DMA for _ in range(NBUF)],
        ],
    )
    def k(x_hbm, idx_hbm, z_hbm, out_hbm, idx_v, bufs, acc, sems):
        c = lax.axis_index("c")
        s = lax.axis_index("s")
        wid = c * NS + s
        he_base = wid * HE_PER_W

        def gather(g, b):
            pltpu.async_copy(
                x_hbm.at[pl.ds(he_base + g * G, G)], bufs[b], sems[b])

        def wait_gather(b):
            pltpu.make_async_copy(x_hbm.at[pl.ds(0, G)], bufs[b], sems[b]).wait()

        def scatter(g, b):
            pltpu.sync_copy(bufs[b], acc.at[idx_v.at[g]], add=True)

        pltpu.sync_copy(idx_hbm.at[wid], idx_v)

        pltpu.sync_copy(z_hbm, bufs[0])
        for j in range(RR):
            cid = s + j * NS

            @pl.when(cid < V_CHUNKS)
            def _():
                pltpu.sync_copy(bufs[0], acc.at[pl.ds(cid * G, G)])
        plsc.subcore_barrier()

        for b in range(NBUF):
            gather(b, b)

        def quad_body(i, _):
            g0 = NBUF * i
            for b in range(NBUF):
                wait_gather(b)
                scatter(g0 + b, b)

                @pl.when(g0 + b + NBUF < GROUPS)
                def _():
                    gather(g0 + b + NBUF, b)
            return 0

        lax.fori_loop(0, GROUPS // NBUF, quad_body, 0)
        for t in range(NBUF * (GROUPS // NBUF), GROUPS):
            wait_gather(t % NBUF)
            scatter(t, t % NBUF)
        plsc.subcore_barrier()

        def out_chunk(cid):
            pltpu.sync_copy(acc.at[pl.ds(cid * G, G)], bufs[1])
            pltpu.sync_copy(bufs[1], out_hbm.at[c, pl.ds(cid * G, G)])

        for j in range(RR):
            cid = s + j * NS
            @pl.when(cid < V_CHUNKS)
            def _():
                out_chunk(cid)

    return k(x, idx3, zrows)


def _combine_body(p_ref, v_ref, o_ref):
    o_ref[...] = (p_ref[0] + p_ref[1]) / v_ref[...]


def _combine(partials, valence):
    rb = 1000
    grid = N_VERT // rb
    return pl.pallas_call(
        _combine_body,
        grid=(grid,),
        in_specs=[
            pl.BlockSpec((NC, rb, D), lambda i: (0, i, 0)),
            pl.BlockSpec((rb, 1), lambda i: (i, 0)),
        ],
        out_specs=pl.BlockSpec((rb, D), lambda i: (i, 0)),
        out_shape=jax.ShapeDtypeStruct((N_VERT, D), jnp.float32),
    )(partials, valence.reshape(N_VERT, 1))


def kernel(x, half_edge_src, vertex_valence):
    idx3 = half_edge_src.astype(jnp.int32).reshape(NW, GROUPS, G)
    zrows = jnp.zeros((G, D), jnp.float32)
    partials = _sc_partial_sums(x, idx3, zrows)
    return _combine(partials, vertex_valence)
